# Initial kernel scaffold; baseline (speedup 1.0000x reference)
import jax
import jax.numpy as jnp
from jax.experimental import pallas as pl


def kernel(coords, embed_x, embed_y, embed_z):
    raise NotImplementedError("write your pallas kernel here")


# speedup vs baseline: 7.0208x; 7.0208x over previous
import functools

import jax
import jax.numpy as jnp
from jax import lax
from jax.experimental import pallas as pl
from jax.experimental.pallas import tpu as pltpu
from jax.experimental.pallas import tpu_sc as plsc

N = 262144
C3 = 128
CH = 3 * C3
NC = 2
NS = 16
NW = NC * NS
PER_W = N // NW
CHUNK = 128
N_CHUNKS = PER_W // CHUNK


def _sc_body(cx_hbm, cy_hbm, cz_hbm, ex_hbm, ey_hbm, ez_hbm, out_hbm,
             idx_v, rows_v, sem_x, sem_y, sem_z):
    cid = lax.axis_index("c")
    sid = lax.axis_index("s")
    base0 = (sid * NC + cid) * PER_W

    def chunk_body(i, carry):
        base = base0 + i * CHUNK
        pltpu.sync_copy(cx_hbm.at[pl.ds(base, CHUNK)], idx_v.at[0])
        pltpu.sync_copy(cy_hbm.at[pl.ds(base, CHUNK)], idx_v.at[1])
        pltpu.sync_copy(cz_hbm.at[pl.ds(base, CHUNK)], idx_v.at[2])
        cx = pltpu.async_copy(ex_hbm.at[idx_v.at[0]],
                              rows_v.at[:, pl.ds(0, C3)], sem_x)
        cy = pltpu.async_copy(ey_hbm.at[idx_v.at[1]],
                              rows_v.at[:, pl.ds(C3, C3)], sem_y)
        cz = pltpu.async_copy(ez_hbm.at[idx_v.at[2]],
                              rows_v.at[:, pl.ds(2 * C3, C3)], sem_z)
        cx.wait()
        cy.wait()
        cz.wait()
        pltpu.sync_copy(rows_v, out_hbm.at[pl.ds(base, CHUNK)])
        return carry

    lax.fori_loop(0, N_CHUNKS, chunk_body, 0)


@functools.partial(jax.jit, static_argnames=())
def kernel(coords, embed_x, embed_y, embed_z):
    cx = coords[:, 0]
    cy = coords[:, 1]
    cz = coords[:, 2]
    mesh = plsc.VectorSubcoreMesh(core_axis_name="c", subcore_axis_name="s")
    run = pl.kernel(
        _sc_body,
        out_type=jax.ShapeDtypeStruct((N, CH), jnp.float32),
        mesh=mesh,
        scratch_types=[
            pltpu.VMEM((3, CHUNK), jnp.int32),
            pltpu.VMEM((CHUNK, CH), jnp.float32),
            pltpu.SemaphoreType.DMA,
            pltpu.SemaphoreType.DMA,
            pltpu.SemaphoreType.DMA,
        ],
    )
    return run(cx, cy, cz, embed_x, embed_y, embed_z)


# speedup vs baseline: 8.5266x; 1.2145x over previous
import jax
import jax.numpy as jnp
from jax import lax
from jax.experimental import pallas as pl
from jax.experimental.pallas import tpu as pltpu
from jax.experimental.pallas import tpu_sc as plsc

N = 262144
C3 = 128
CH = 3 * C3
NC = 2
NS = 16
NW = NC * NS
PER_W = N // NW
CHUNK = 128
N_CHUNKS = PER_W // CHUNK


def _sc_body(cx_hbm, cy_hbm, cz_hbm, ex_hbm, ey_hbm, ez_hbm, out_hbm,
             idx_v, rows_v, gsem0, gsem1, wsem0, wsem1):
    cid = lax.axis_index("c")
    sid = lax.axis_index("s")
    base0 = (sid * NC + cid) * PER_W
    gsem = (gsem0, gsem1)
    wsem = (wsem0, wsem1)
    tables = (ex_hbm, ey_hbm, ez_hbm)
    coords = (cx_hbm, cy_hbm, cz_hbm)

    def fire_gathers(i, b):
        base = base0 + i * CHUNK
        for d in range(3):
            pltpu.sync_copy(coords[d].at[pl.ds(base, CHUNK)], idx_v.at[b, d])
        for d in range(3):
            pltpu.async_copy(tables[d].at[idx_v.at[b, d]],
                             rows_v.at[b, :, pl.ds(d * C3, C3)], gsem[b])

    def wait_gathers(b):
        for d in range(3):
            pltpu.make_async_copy(tables[d].at[idx_v.at[b, d]],
                                  rows_v.at[b, :, pl.ds(d * C3, C3)],
                                  gsem[b]).wait()

    def fire_write(i, b):
        base = base0 + i * CHUNK
        pltpu.async_copy(rows_v.at[b], out_hbm.at[pl.ds(base, CHUNK)], wsem[b])

    def wait_write(b):
        pltpu.make_async_copy(rows_v.at[b], out_hbm.at[pl.ds(base0, CHUNK)],
                              wsem[b]).wait()

    fire_gathers(0, 0)

    def pair_body(g, carry):
        @pl.when(g >= 1)
        def _():
            wait_write(0)
            fire_gathers(2 * g, 0)
            wait_gathers(1)
            fire_write(2 * g - 1, 1)

        @pl.when(g >= 1)
        def _():
            wait_write(1)
        fire_gathers(2 * g + 1, 1)
        wait_gathers(0)
        fire_write(2 * g, 0)
        return carry

    lax.fori_loop(0, N_CHUNKS // 2, pair_body, 0)

    wait_gathers(1)
    fire_write(N_CHUNKS - 1, 1)
    wait_write(0)
    wait_write(1)


def kernel(coords, embed_x, embed_y, embed_z):
    cx = coords[:, 0]
    cy = coords[:, 1]
    cz = coords[:, 2]
    mesh = plsc.VectorSubcoreMesh(core_axis_name="c", subcore_axis_name="s")
    run = pl.kernel(
        _sc_body,
        out_type=jax.ShapeDtypeStruct((N, CH), jnp.float32),
        mesh=mesh,
        scratch_types=[
            pltpu.VMEM((2, 3, CHUNK), jnp.int32),
            pltpu.VMEM((2, CHUNK, CH), jnp.float32),
            pltpu.SemaphoreType.DMA,
            pltpu.SemaphoreType.DMA,
            pltpu.SemaphoreType.DMA,
            pltpu.SemaphoreType.DMA,
        ],
    )
    return run(cx, cy, cz, embed_x, embed_y, embed_z)


# speedup vs baseline: 9.0754x; 1.0644x over previous
import jax
import jax.numpy as jnp
from jax import lax
from jax.experimental import pallas as pl
from jax.experimental.pallas import tpu as pltpu
from jax.experimental.pallas import tpu_sc as plsc

N = 262144
C3 = 128
CH = 3 * C3
NC = 2
NS = 16
NW = NC * NS
PER_W = N // NW
CHUNK = 128
N_CHUNKS = PER_W // CHUNK


def _sc_body(cx_hbm, cy_hbm, cz_hbm, ex_hbm, ey_hbm, ez_hbm, out_hbm,
             ix_v, iy_v, iz_v, rows_v, gsem0, gsem1, wsem0, wsem1):
    cid = lax.axis_index("c")
    sid = lax.axis_index("s")
    base0 = (sid * NC + cid) * PER_W
    gsem = (gsem0, gsem1)
    wsem = (wsem0, wsem1)
    tables = (ex_hbm, ey_hbm, ez_hbm)
    coords = (cx_hbm, cy_hbm, cz_hbm)

    idx_v = (ix_v, iy_v, iz_v)
    for d in range(3):
        pltpu.sync_copy(coords[d].at[pl.ds(base0, PER_W)], idx_v[d])

    def fire_gathers(i, b):
        for d in range(3):
            pltpu.async_copy(tables[d].at[idx_v[d].at[pl.ds(i * CHUNK, CHUNK)]],
                             rows_v.at[b, :, pl.ds(d * C3, C3)], gsem[b])

    def wait_gathers(b):
        for d in range(3):
            pltpu.make_async_copy(tables[d].at[idx_v[d].at[pl.ds(0, CHUNK)]],
                                  rows_v.at[b, :, pl.ds(d * C3, C3)],
                                  gsem[b]).wait()

    def fire_write(i, b):
        base = base0 + i * CHUNK
        pltpu.async_copy(rows_v.at[b], out_hbm.at[pl.ds(base, CHUNK)], wsem[b])

    def wait_write(b):
        pltpu.make_async_copy(rows_v.at[b], out_hbm.at[pl.ds(base0, CHUNK)],
                              wsem[b]).wait()

    fire_gathers(0, 0)

    def pair_body(g, carry):
        @pl.when(g >= 1)
        def _():
            wait_write(0)
            fire_gathers(2 * g, 0)
            wait_gathers(1)
            fire_write(2 * g - 1, 1)

        @pl.when(g >= 1)
        def _():
            wait_write(1)
        fire_gathers(2 * g + 1, 1)
        wait_gathers(0)
        fire_write(2 * g, 0)
        return carry

    lax.fori_loop(0, N_CHUNKS // 2, pair_body, 0)

    wait_gathers(1)
    fire_write(N_CHUNKS - 1, 1)
    wait_write(0)
    wait_write(1)


def kernel(coords, embed_x, embed_y, embed_z):
    cx = coords[:, 0]
    cy = coords[:, 1]
    cz = coords[:, 2]
    mesh = plsc.VectorSubcoreMesh(core_axis_name="c", subcore_axis_name="s")
    run = pl.kernel(
        _sc_body,
        out_type=jax.ShapeDtypeStruct((N, CH), jnp.float32),
        mesh=mesh,
        scratch_types=[
            pltpu.VMEM((PER_W,), jnp.int32),
            pltpu.VMEM((PER_W,), jnp.int32),
            pltpu.VMEM((PER_W,), jnp.int32),
            pltpu.VMEM((2, CHUNK, CH), jnp.float32),
            pltpu.SemaphoreType.DMA,
            pltpu.SemaphoreType.DMA,
            pltpu.SemaphoreType.DMA,
            pltpu.SemaphoreType.DMA,
        ],
    )
    return run(cx, cy, cz, embed_x, embed_y, embed_z)
